# baseline (speedup 1.0000x reference)
import jax
import jax.numpy as jnp
from jax import lax
from jax.experimental import pallas as pl
from jax.experimental.pallas import tpu as pltpu
from jax.experimental.pallas import tpu_sc as plsc

B = 4096
L = 200
D = 32
V = 1000000

NC = 2
NS = 16
NW = NC * NS
RPW = B // NW
LANES = 16
C0 = 128
C1 = L - C0

_GDN = lax.GatherDimensionNumbers(
    offset_dims=(), collapsed_slice_dims=(0,), start_index_map=(0,))


def _shuffle(v, idx):
    return lax.gather(v, idx[:, None], _GDN, slice_sizes=(1,),
                      mode=lax.GatherScatterMode.PROMISE_IN_BOUNDS)


def _lane_reduce(v, op):
    lane = lax.iota(jnp.int32, LANES)
    for sh in (8, 4, 2, 1):
        v = op(v, _shuffle(v, lane ^ sh))
    return v


_CHUNKS = [(LANES * k, 0) for k in range(12)] + [(L - LANES, 8)]


def _copies(embed_hbm, wt_hbm, idx_blk, r, rows_v, w_v, sem):
    return [
        pltpu.make_async_copy(embed_hbm.at[idx_blk.at[r, pl.ds(0, C0)]],
                              rows_v.at[pl.ds(0, C0)], sem),
        pltpu.make_async_copy(embed_hbm.at[idx_blk.at[r, pl.ds(C0, C1)]],
                              rows_v.at[pl.ds(C0, C1)], sem),
        pltpu.make_async_copy(wt_hbm.at[idx_blk.at[r, pl.ds(0, C0)]],
                              w_v.at[pl.ds(0, C0)], sem),
        pltpu.make_async_copy(wt_hbm.at[idx_blk.at[r, pl.ds(C0, C1)]],
                              w_v.at[pl.ds(C0, C1)], sem),
    ]


def _fire(*args):
    for cp in _copies(*args):
        cp.start()


def _drain(*args):
    for cp in _copies(*args):
        cp.wait()


def _compute_row(r, idx_blk, rows_v, w_v, out_blk):
    lane = lax.iota(jnp.int32, LANES)

    m_vec = jnp.full((LANES,), -1e30, jnp.float32)
    for base_c, v0 in _CHUNKS:
        wc = w_v[pl.ds(base_c, LANES)]
        if v0:
            wc = jnp.where(lane >= v0, wc, -1e30)
        m_vec = jnp.maximum(m_vec, wc)
    m = _lane_reduce(m_vec, jnp.maximum)

    s_vec = jnp.zeros((LANES,), jnp.float32)
    acc0 = jnp.zeros((LANES,), jnp.float32)
    acc1 = jnp.zeros((LANES,), jnp.float32)
    for base_c, v0 in _CHUNKS:
        p_vec = jnp.exp(w_v[pl.ds(base_c, LANES)] - m)
        if v0:
            p_vec = jnp.where(lane >= v0, p_vec, 0.0)
        s_vec = s_vec + p_vec
        for j in range(v0, LANES):
            pj = p_vec[j]
            tok = base_c + j
            acc0 = acc0 + pj * rows_v[tok, pl.ds(0, LANES)]
            acc1 = acc1 + pj * rows_v[tok, pl.ds(LANES, LANES)]

    inv = 1.0 / _lane_reduce(s_vec, jnp.add)
    out_blk[r, pl.ds(0, LANES)] = acc0 * inv
    out_blk[r, pl.ds(LANES, LANES)] = acc1 * inv


def _body(doc_hbm, embed_hbm, wt_hbm, out_hbm, idx_blk,
          rows_a, rows_b, w_a, w_b, out_blk, sem_a, sem_b):
    cid = lax.axis_index("c")
    sid = lax.axis_index("s")
    wid = sid * NC + cid
    base = wid * RPW

    pltpu.sync_copy(doc_hbm.at[pl.ds(base, RPW), :], idx_blk)

    _fire(embed_hbm, wt_hbm, idx_blk, 0, rows_a, w_a, sem_a)

    def pair_body(i, carry):
        r0 = 2 * i
        _fire(embed_hbm, wt_hbm, idx_blk, r0 + 1, rows_b, w_b, sem_b)
        _drain(embed_hbm, wt_hbm, idx_blk, r0, rows_a, w_a, sem_a)
        _compute_row(r0, idx_blk, rows_a, w_a, out_blk)

        @pl.when(i < RPW // 2 - 1)
        def _():
            _fire(embed_hbm, wt_hbm, idx_blk, r0 + 2, rows_a, w_a, sem_a)

        _drain(embed_hbm, wt_hbm, idx_blk, r0 + 1, rows_b, w_b, sem_b)
        _compute_row(r0 + 1, idx_blk, rows_b, w_b, out_blk)
        return carry

    lax.fori_loop(0, RPW // 2, pair_body, 0)
    pltpu.sync_copy(out_blk, out_hbm.at[pl.ds(base, RPW), :])


@jax.jit
def _doc_encode(document, embed_table, wt_flat):
    f = pl.kernel(
        _body,
        out_type=jax.ShapeDtypeStruct((B, D), jnp.float32),
        mesh=plsc.VectorSubcoreMesh(core_axis_name="c", subcore_axis_name="s"),
        compiler_params=pltpu.CompilerParams(use_tc_tiling_on_sc=False),
        scratch_types=[
            pltpu.VMEM((RPW, L), jnp.int32),
            pltpu.VMEM((L, D), jnp.float32),
            pltpu.VMEM((L, D), jnp.float32),
            pltpu.VMEM((L,), jnp.float32),
            pltpu.VMEM((L,), jnp.float32),
            pltpu.VMEM((RPW, D), jnp.float32),
            pltpu.SemaphoreType.DMA,
            pltpu.SemaphoreType.DMA,
        ],
    )
    return f(document, embed_table, wt_flat)


def kernel(document, lens, embed_table, weight_table):
    del lens
    wt_flat = jnp.pad(weight_table, ((0, 64), (0, 0))).reshape((V + 64,))
    return _doc_encode(document, embed_table, wt_flat)


# speedup vs baseline: 1.0209x; 1.0209x over previous
import jax
import jax.numpy as jnp
from jax import lax
from jax.experimental import pallas as pl
from jax.experimental.pallas import tpu as pltpu
from jax.experimental.pallas import tpu_sc as plsc

B = 4096
L = 200
D = 32
V = 1000000

NC = 2
NS = 16
NW = NC * NS
RPW = B // NW
LANES = 16
C0 = 128
C1 = L - C0

_GDN = lax.GatherDimensionNumbers(
    offset_dims=(), collapsed_slice_dims=(0,), start_index_map=(0,))


def _shuffle(v, idx):
    return lax.gather(v, idx[:, None], _GDN, slice_sizes=(1,),
                      mode=lax.GatherScatterMode.PROMISE_IN_BOUNDS)


def _lane_reduce(v, op):
    lane = lax.iota(jnp.int32, LANES)
    for sh in (8, 4, 2, 1):
        v = op(v, _shuffle(v, lane ^ sh))
    return v


_CHUNKS = [(LANES * k, 0) for k in range(12)] + [(L - LANES, 8)]


def _w_copies(wt_hbm, idx_blk, r, w_v, sem):
    return [
        pltpu.make_async_copy(wt_hbm.at[idx_blk.at[r, pl.ds(0, C0)]],
                              w_v.at[pl.ds(0, C0)], sem),
        pltpu.make_async_copy(wt_hbm.at[idx_blk.at[r, pl.ds(C0, C1)]],
                              w_v.at[pl.ds(C0, C1)], sem),
    ]


def _e_copies(embed_hbm, idx_blk, r, rows_v, sem):
    return [
        pltpu.make_async_copy(embed_hbm.at[idx_blk.at[r, pl.ds(0, C0)]],
                              rows_v.at[pl.ds(0, C0)], sem),
        pltpu.make_async_copy(embed_hbm.at[idx_blk.at[r, pl.ds(C0, C1)]],
                              rows_v.at[pl.ds(C0, C1)], sem),
    ]


def _softmax_row(r, w_v, p_blk):
    lane = lax.iota(jnp.int32, LANES)
    m_vec = jnp.full((LANES,), -1e30, jnp.float32)
    for base_c, v0 in _CHUNKS:
        wc = w_v[pl.ds(base_c, LANES)]
        if v0:
            wc = jnp.where(lane >= v0, wc, -1e30)
        m_vec = jnp.maximum(m_vec, wc)
    m = _lane_reduce(m_vec, jnp.maximum)

    s_vec = jnp.zeros((LANES,), jnp.float32)
    ps = []
    for base_c, v0 in _CHUNKS:
        p_vec = jnp.exp(w_v[pl.ds(base_c, LANES)] - m)
        if v0:
            p_vec = jnp.where(lane >= v0, p_vec, 0.0)
        s_vec = s_vec + p_vec
        ps.append(p_vec)
    inv = 1.0 / _lane_reduce(s_vec, jnp.add)
    order = [len(_CHUNKS) - 1] + list(range(len(_CHUNKS) - 1))
    for k in order:
        base_c, _ = _CHUNKS[k]
        p_blk[r, pl.ds(base_c, LANES)] = ps[k] * inv


def _body_a(doc_hbm, wt_hbm, p_hbm, idx_blk, w_a, w_b, p_blk, sem_a, sem_b):
    cid = lax.axis_index("c")
    sid = lax.axis_index("s")
    wid = sid * NC + cid
    base = wid * RPW
    pltpu.sync_copy(doc_hbm.at[pl.ds(base, RPW), :], idx_blk)

    for cp in _w_copies(wt_hbm, idx_blk, 0, w_a, sem_a):
        cp.start()

    def pair_body(i, carry):
        r0 = 2 * i
        for cp in _w_copies(wt_hbm, idx_blk, r0 + 1, w_b, sem_b):
            cp.start()
        for cp in _w_copies(wt_hbm, idx_blk, r0, w_a, sem_a):
            cp.wait()
        _softmax_row(r0, w_a, p_blk)

        @pl.when(i < RPW // 2 - 1)
        def _():
            for cp in _w_copies(wt_hbm, idx_blk, r0 + 2, w_a, sem_a):
                cp.start()

        for cp in _w_copies(wt_hbm, idx_blk, r0 + 1, w_b, sem_b):
            cp.wait()
        _softmax_row(r0 + 1, w_b, p_blk)
        return carry

    lax.fori_loop(0, RPW // 2, pair_body, 0)
    pltpu.sync_copy(p_blk, p_hbm.at[pl.ds(base, RPW), :])


def _sum_row(r, rows_v, p_blk, out_blk):
    acc0 = jnp.zeros((LANES,), jnp.float32)
    acc1 = jnp.zeros((LANES,), jnp.float32)
    for base_c, v0 in _CHUNKS:
        p_vec = p_blk[r, pl.ds(base_c, LANES)]
        for j in range(v0, LANES):
            pj = p_vec[j]
            tok = base_c + j
            acc0 = acc0 + pj * rows_v[tok, pl.ds(0, LANES)]
            acc1 = acc1 + pj * rows_v[tok, pl.ds(LANES, LANES)]
    out_blk[r, pl.ds(0, LANES)] = acc0
    out_blk[r, pl.ds(LANES, LANES)] = acc1


def _body_b(doc_hbm, embed_hbm, p_hbm, out_hbm,
            idx_blk, rows_a, rows_b, p_blk, out_blk, sem_a, sem_b):
    cid = lax.axis_index("c")
    sid = lax.axis_index("s")
    wid = sid * NC + cid
    base = wid * RPW
    pltpu.sync_copy(doc_hbm.at[pl.ds(base, RPW), :], idx_blk)
    pltpu.sync_copy(p_hbm.at[pl.ds(base, RPW), :], p_blk)

    for cp in _e_copies(embed_hbm, idx_blk, 0, rows_a, sem_a):
        cp.start()

    def pair_body(i, carry):
        r0 = 2 * i
        for cp in _e_copies(embed_hbm, idx_blk, r0 + 1, rows_b, sem_b):
            cp.start()
        for cp in _e_copies(embed_hbm, idx_blk, r0, rows_a, sem_a):
            cp.wait()
        _sum_row(r0, rows_a, p_blk, out_blk)

        @pl.when(i < RPW // 2 - 1)
        def _():
            for cp in _e_copies(embed_hbm, idx_blk, r0 + 2, rows_a, sem_a):
                cp.start()

        for cp in _e_copies(embed_hbm, idx_blk, r0 + 1, rows_b, sem_b):
            cp.wait()
        _sum_row(r0 + 1, rows_b, p_blk, out_blk)
        return carry

    lax.fori_loop(0, RPW // 2, pair_body, 0)
    pltpu.sync_copy(out_blk, out_hbm.at[pl.ds(base, RPW), :])


@jax.jit
def _doc_encode(document, embed_table, wt_flat):
    fa = pl.kernel(
        _body_a,
        out_type=jax.ShapeDtypeStruct((B, L), jnp.float32),
        mesh=plsc.VectorSubcoreMesh(core_axis_name="c", subcore_axis_name="s"),
        compiler_params=pltpu.CompilerParams(use_tc_tiling_on_sc=False),
        scratch_types=[
            pltpu.VMEM((RPW, L), jnp.int32),
            pltpu.VMEM((L,), jnp.float32),
            pltpu.VMEM((L,), jnp.float32),
            pltpu.VMEM((RPW, L), jnp.float32),
            pltpu.SemaphoreType.DMA,
            pltpu.SemaphoreType.DMA,
        ],
    )
    p = fa(document, wt_flat)
    fb = pl.kernel(
        _body_b,
        out_type=jax.ShapeDtypeStruct((B, D), jnp.float32),
        mesh=plsc.VectorSubcoreMesh(core_axis_name="c", subcore_axis_name="s"),
        compiler_params=pltpu.CompilerParams(use_tc_tiling_on_sc=False),
        scratch_types=[
            pltpu.VMEM((RPW, L), jnp.int32),
            pltpu.VMEM((L, D), jnp.float32),
            pltpu.VMEM((L, D), jnp.float32),
            pltpu.VMEM((RPW, L), jnp.float32),
            pltpu.VMEM((RPW, D), jnp.float32),
            pltpu.SemaphoreType.DMA,
            pltpu.SemaphoreType.DMA,
        ],
    )
    return fb(document, embed_table, p)


def kernel(document, lens, embed_table, weight_table):
    del lens
    wt_flat = jnp.pad(weight_table, ((0, 64), (0, 0))).reshape((V + 64,))
    return _doc_encode(document, embed_table, wt_flat)
